# Initial kernel scaffold; baseline (speedup 1.0000x reference)
import jax
import jax.numpy as jnp
from jax.experimental import pallas as pl


def kernel(x_nchw, w1, g1, b1, w2, g2, b2):
    raise NotImplementedError("write your pallas kernel here")


# speedup vs baseline: 4.5439x; 4.5439x over previous
import jax
import jax.numpy as jnp
from jax.experimental import pallas as pl
from jax.experimental.pallas import tpu as pltpu

_EPS = 1e-5
_SLOPE = 0.1
_MM = jnp.bfloat16


def _conv_stats_from_patches(acc, y_ref, s_ref, q_ref):
    H, W, C = y_ref.shape[1], y_ref.shape[2], y_ref.shape[3]
    y_ref[0] = acc.reshape(H, W, C)
    s_ref[0] = jnp.sum(acc, axis=0, keepdims=True)
    q_ref[0] = jnp.sum(acc * acc, axis=0, keepdims=True)


def _conv1_kernel(xc_ref, w_ref, y_ref, s_ref, q_ref):
    H = xc_ref.shape[1] - 2
    W = xc_ref.shape[2]
    K = xc_ref.shape[3]
    acc = None
    for dy in range(3):
        slab = xc_ref[0, dy:dy + H].reshape(H * W, K)
        d = jnp.dot(slab, w_ref[dy], preferred_element_type=jnp.float32)
        acc = d if acc is None else acc + d
    _conv_stats_from_patches(acc, y_ref, s_ref, q_ref)


def _mid_kernel(y1_ref, s_ref, q_ref, g_ref, b_ref, w_ref,
                y2_ref, s2_ref, q2_ref, hc_ref):
    N = s_ref.shape[0]
    H, W, C1 = y1_ref.shape[1], y1_ref.shape[2], y1_ref.shape[3]
    R = float(N * H * W)

    mean = jnp.sum(s_ref[...], axis=0) / R
    var = jnp.sum(q_ref[...], axis=0) / R - mean * mean
    scale = g_ref[...] * jax.lax.rsqrt(var + _EPS)
    shift = b_ref[...] - mean * scale

    h = y1_ref[0] * scale.reshape(1, 1, C1) + shift.reshape(1, 1, C1)
    h = jnp.where(h >= 0.0, h, _SLOPE * h)
    hp = jnp.pad(h, ((0, 0), (1, 1), (0, 0))).astype(_MM)

    hc_ref[0] = jnp.zeros((W, 3 * C1), _MM)
    hc_ref[H + 1] = jnp.zeros((W, 3 * C1), _MM)
    for dx in range(3):
        hc_ref[1:H + 1, :, dx * C1:(dx + 1) * C1] = hp[:, dx:dx + W, :]

    acc = None
    for dy in range(3):
        slab = hc_ref[dy:dy + H].reshape(H * W, 3 * C1)
        d = jnp.dot(slab, w_ref[dy], preferred_element_type=jnp.float32)
        acc = d if acc is None else acc + d
    _conv_stats_from_patches(acc, y2_ref, s2_ref, q2_ref)


def _out_kernel(y2_ref, s_ref, q_ref, g_ref, b_ref, o_ref):
    N = s_ref.shape[0]
    H, W, C = y2_ref.shape[1], y2_ref.shape[2], y2_ref.shape[3]
    R = float(N * H * W)
    mean = jnp.sum(s_ref[...], axis=0) / R
    var = jnp.sum(q_ref[...], axis=0) / R - mean * mean
    scale = g_ref[...] * jax.lax.rsqrt(var + _EPS)
    shift = b_ref[...] - mean * scale
    h = y2_ref[0] * scale.reshape(1, 1, C) + shift.reshape(1, 1, C)
    o_ref[0] = jnp.where(h >= 0.0, h, _SLOPE * h)


@jax.jit
def _double_conv(x_nchw, w1, g1, b1, w2, g2, b2):
    N, C1, H, W = x_nchw.shape
    Cm = w1.shape[-1]
    C2 = w2.shape[-1]

    x = jnp.transpose(x_nchw, (0, 2, 3, 1)).astype(_MM)
    xp = jnp.pad(x, ((0, 0), (1, 1), (1, 1), (0, 0)))
    xc = jnp.concatenate(
        [xp[:, :, 0:W], xp[:, :, 1:W + 1], xp[:, :, 2:W + 2]], axis=-1)
    w1c = w1.reshape(3, 3 * C1, Cm).astype(_MM)
    w2c = w2.reshape(3, 3 * Cm, C2).astype(_MM)
    g1r = g1.astype(jnp.float32).reshape(1, Cm)
    b1r = b1.astype(jnp.float32).reshape(1, Cm)
    g2r = g2.astype(jnp.float32).reshape(1, C2)
    b2r = b2.astype(jnp.float32).reshape(1, C2)

    vec = lambda c: pl.BlockSpec((1, c), lambda i: (0, 0))
    stat_in = lambda c: pl.BlockSpec((N, 1, c), lambda i: (0, 0, 0))
    stat_out = lambda c: pl.BlockSpec((1, 1, c), lambda i: (i, 0, 0))
    img = lambda c: pl.BlockSpec((1, H, W, c), lambda i: (i, 0, 0, 0))

    y1, s1, q1 = pl.pallas_call(
        _conv1_kernel,
        grid=(N,),
        in_specs=[pl.BlockSpec((1, H + 2, W, 3 * C1), lambda i: (i, 0, 0, 0)),
                  pl.BlockSpec((3, 3 * C1, Cm), lambda i: (0, 0, 0))],
        out_specs=[img(Cm), stat_out(Cm), stat_out(Cm)],
        out_shape=[jax.ShapeDtypeStruct((N, H, W, Cm), jnp.float32),
                   jax.ShapeDtypeStruct((N, 1, Cm), jnp.float32),
                   jax.ShapeDtypeStruct((N, 1, Cm), jnp.float32)],
        compiler_params=pltpu.CompilerParams(
            dimension_semantics=("parallel",)),
    )(xc, w1c)

    y2, s2, q2 = pl.pallas_call(
        _mid_kernel,
        grid=(N,),
        in_specs=[img(Cm), stat_in(Cm), stat_in(Cm), vec(Cm), vec(Cm),
                  pl.BlockSpec((3, 3 * Cm, C2), lambda i: (0, 0, 0))],
        out_specs=[img(C2), stat_out(C2), stat_out(C2)],
        out_shape=[jax.ShapeDtypeStruct((N, H, W, C2), jnp.float32),
                   jax.ShapeDtypeStruct((N, 1, C2), jnp.float32),
                   jax.ShapeDtypeStruct((N, 1, C2), jnp.float32)],
        scratch_shapes=[pltpu.VMEM((H + 2, W, 3 * Cm), _MM)],
        compiler_params=pltpu.CompilerParams(
            dimension_semantics=("parallel",)),
    )(y1, s1, q1, g1r, b1r, w2c)

    out = pl.pallas_call(
        _out_kernel,
        grid=(N,),
        in_specs=[img(C2), stat_in(C2), stat_in(C2), vec(C2), vec(C2)],
        out_specs=img(C2),
        out_shape=jax.ShapeDtypeStruct((N, H, W, C2), jnp.float32),
        compiler_params=pltpu.CompilerParams(
            dimension_semantics=("parallel",)),
    )(y2, s2, q2, g2r, b2r)

    return jnp.transpose(out, (0, 3, 1, 2))


def kernel(x_nchw, w1, g1, b1, w2, g2, b2):
    return _double_conv(x_nchw, w1, g1, b1, w2, g2, b2)
